# baseline (speedup 1.0000x reference)
import functools

import jax
import jax.numpy as jnp
from jax.experimental import pallas as pl
from jax.experimental.pallas import tpu as pltpu
from jax.experimental.pallas import tpu_sc as plsc

N = 10000
E = 320000
D = 128
H = 128
C = 40
CP = 48

NC = 2
NS = 16
NW = NC * NS
NPAD = 10240
ROWS_W = NPAD // NW
ROWS_T = NPAD // NS
BLK = 512

EPW = 10240
EPAD = EPW * NW
PAD_ROW = N + 100

EBLK = 2000
NEB = E // EBLK

_SC_MESH = plsc.VectorSubcoreMesh(
    core_axis_name="c", subcore_axis_name="s", num_cores=NC, num_subcores=NS)


EPT = E // NW


@functools.partial(
    pl.kernel,
    out_type=jax.ShapeDtypeStruct((NC, NPAD), jnp.float32),
    mesh=_SC_MESH,
    scratch_types=[
        pltpu.VMEM((EPT,), jnp.int32),
        pltpu.VMEM((NPAD,), jnp.float32),
        pltpu.VMEM((NS, ROWS_T), jnp.float32),
        pltpu.VMEM_SHARED((NS, NPAD), jnp.float32),
    ],
    compiler_params=pltpu.CompilerParams(needs_layout_passes=False),
)
def _deg_kernel(dst_hbm, deg_hbm, dslice, hist, rbuf, stage):
    c = jax.lax.axis_index("c")
    s = jax.lax.axis_index("s")
    wid = s * NC + c

    zeros16 = jnp.zeros((16,), jnp.float32)

    def zbody(i, _):
        hist[pl.ds(i * 16, 16)] = zeros16
        return 0

    jax.lax.fori_loop(0, NPAD // 16, zbody, 0)

    off = pl.multiple_of(wid * EPT, 8)
    pltpu.sync_copy(dst_hbm.at[pl.ds(off, EPT)], dslice)

    ones16 = jnp.ones((16,), jnp.float32)

    def in_body(i, _):
        d16 = dslice[pl.ds(i * 16, 16)]
        plsc.addupdate_scatter(hist, [d16], ones16)
        return 0

    jax.lax.fori_loop(0, EPT // 16, in_body, 0)

    pltpu.sync_copy(hist, stage.at[s])
    plsc.subcore_barrier()

    row0 = s * ROWS_T
    pltpu.sync_copy(stage.at[:, pl.ds(row0, ROWS_T)], rbuf)

    def red_body(g, _):
        acc = rbuf[0, pl.ds(g * 16, 16)]
        for p in range(1, NS):
            acc = acc + rbuf[p, pl.ds(g * 16, 16)]
        hist[pl.ds(g * 16, 16)] = acc
        return 0

    jax.lax.fori_loop(0, ROWS_T // 16, red_body, 0)
    pltpu.sync_copy(hist.at[pl.ds(0, ROWS_T)],
                    deg_hbm.at[c, pl.ds(row0, ROWS_T)])


def _scat_body(srcp, dstp, hp, out_hbm,
               src0, src1, src2, src3, dst0, dst1, dst2, dst3,
               rows0, rows1, rows2, rows3, acc,
               gs0, gs1, gs2, gs3, ss0, ss1, ss2, ss3,
               is0, is1, is2, is3, *, F, CH, NCH):
    c = jax.lax.axis_index("c")
    s = jax.lax.axis_index("s")
    wid = s * NC + c

    srcv = [src0, src1, src2, src3]
    dstv = [dst0, dst1, dst2, dst3]
    rows = [rows0, rows1, rows2, rows3]
    gsem = [gs0, gs1, gs2, gs3]
    ssem = [ss0, ss1, ss2, ss3]
    isem = [is0, is1, is2, is3]

    def idx_start(j, t):
        pltpu.async_copy(srcp.at[wid, j], srcv[t], isem[t])
        pltpu.async_copy(dstp.at[wid, j], dstv[t], isem[t])

    def idx_wait(j, t):
        pltpu.make_async_copy(srcp.at[wid, j], srcv[t], isem[t]).wait()
        pltpu.make_async_copy(dstp.at[wid, j], dstv[t], isem[t]).wait()

    def gather_start(t):
        pltpu.async_copy(hp.at[srcv[t]], rows[t], gsem[t])

    def gather_wait(t):
        pltpu.make_async_copy(hp.at[srcv[t]], rows[t], gsem[t]).wait()

    def scat_start(t):
        pltpu.async_copy(rows[t], acc.at[dstv[t]], ssem[t], add=True)

    def scat_wait(t):
        pltpu.make_async_copy(rows[t], acc.at[dstv[t]], ssem[t]).wait()

    zeros16 = jnp.zeros((16,), jnp.float32)

    def zrow(i, _):
        def zcol(g, _):
            rows0[i, pl.ds(g * 16, 16)] = zeros16
            return 0
        jax.lax.fori_loop(0, F // 16, zcol, 0)
        return 0

    jax.lax.fori_loop(0, CH, zrow, 0)

    row0 = s * ROWS_T

    def zacc(b, _):
        pltpu.sync_copy(rows0, acc.at[pl.ds(row0 + b * CH, CH)])
        return 0

    jax.lax.fori_loop(0, ROWS_T // CH, zacc, 0)
    plsc.subcore_barrier()

    idx_start(0, 0)
    idx_start(1, 1)
    idx_wait(0, 0)
    gather_start(0)
    idx_wait(1, 1)
    gather_start(1)

    def body(k, _):
        for t in range(4):
            b = 4 * k + t
            u = (t + 2) % 4
            gather_wait(t)
            scat_start(t)

            @pl.when(b >= 2)
            def _():
                scat_wait(u)

            @pl.when(b + 2 < NCH)
            def _():
                idx_start(b + 2, u)
                idx_wait(b + 2, u)
                gather_start(u)

        return 0

    jax.lax.fori_loop(0, NCH // 4, body, 0)
    scat_wait((NCH - 2) % 4)
    scat_wait((NCH - 1) % 4)
    plsc.subcore_barrier()

    pltpu.sync_copy(acc.at[pl.ds(row0, ROWS_T)],
                    out_hbm.at[c, pl.ds(row0, ROWS_T)])


def _make_scat(F, CH):
    NCH = EPW // CH
    return functools.partial(
        pl.kernel,
        out_type=jax.ShapeDtypeStruct((NC, NPAD, F), jnp.float32),
        mesh=_SC_MESH,
        scratch_types=(
            [pltpu.VMEM((CH,), jnp.int32) for _ in range(8)]
            + [pltpu.VMEM((CH, F), jnp.float32) for _ in range(4)]
            + [pltpu.VMEM_SHARED((NPAD, F), jnp.float32)]
            + [pltpu.SemaphoreType.DMA for _ in range(12)]
        ),
        compiler_params=pltpu.CompilerParams(use_tc_tiling_on_sc=False),
    )(functools.partial(_scat_body, F=F, CH=CH, NCH=NCH))


CH_H = 80
CH_C = 80
_scat_h = _make_scat(H, CH_H)
_scat_c = _make_scat(CP, CH_C)


def _mm1_body(x_ref, w1_ref, deg_ref, hp1_ref, dinv_ref):
    deg = deg_ref[0] + deg_ref[1] + 1.0
    dinv = jax.lax.rsqrt(deg)
    h = jnp.dot(x_ref[...], w1_ref[...], preferred_element_type=jnp.float32)
    hp1_ref[...] = h * dinv[:, None]
    dinv_ref[...] = dinv


def _mm2_body(s1_ref, hp1_ref, dinv_ref, b1_ref, w2_ref, x1_ref, hp2_ref):
    dinv = dinv_ref[...]
    x1 = (s1_ref[0] + s1_ref[1] + hp1_ref[...]) * dinv[:, None] + b1_ref[...]
    x1_ref[...] = x1
    x2 = jnp.maximum(x1, 0.0)
    h2 = jnp.dot(x2, w2_ref[...], preferred_element_type=jnp.float32)
    hp2_ref[...] = h2 * dinv[:, None]


def _fin_body(s2_ref, hp2_ref, dinv_ref, b2_ref, out_ref):
    dinv = dinv_ref[...]
    out_ref[...] = (s2_ref[0] + s2_ref[1] + hp2_ref[...]) * dinv[:, None] \
        + b2_ref[...]


def _row_spec(cols):
    return pl.BlockSpec((BLK, cols), lambda i: (i, 0))


def _part_spec(cols):
    return pl.BlockSpec((NC, BLK, cols), lambda i: (0, i, 0))


def _vec_spec():
    return pl.BlockSpec((BLK,), lambda i: (i,))


def _full_spec(r, co):
    return pl.BlockSpec((r, co), lambda i: (0, 0))


def kernel(last_e_emb, edge_index, W1, b1, W2, b2):
    src = edge_index[0]
    dst = edge_index[1]

    xp = jnp.zeros((NPAD, D), jnp.float32).at[:N].set(last_e_emb)

    pad = jnp.full((EPAD - E,), PAD_ROW, jnp.int32)
    srcf = jnp.concatenate([src, pad])
    dstf = jnp.concatenate([dst, pad])
    srcp_h = srcf.reshape(NW, EPW // CH_H, CH_H)
    dstp_h = dstf.reshape(NW, EPW // CH_H, CH_H)
    srcp_c = srcf.reshape(NW, EPW // CH_C, CH_C)
    dstp_c = dstf.reshape(NW, EPW // CH_C, CH_C)

    deg = _deg_kernel(dst)

    grid = (NPAD // BLK,)
    hp1, dinv = pl.pallas_call(
        _mm1_body,
        grid=grid,
        in_specs=[_row_spec(D), _full_spec(D, H),
                  pl.BlockSpec((NC, BLK), lambda i: (0, i))],
        out_specs=[_row_spec(H), _vec_spec()],
        out_shape=[jax.ShapeDtypeStruct((NPAD, H), jnp.float32),
                   jax.ShapeDtypeStruct((NPAD,), jnp.float32)],
    )(xp, W1, deg)

    s1 = _scat_h(srcp_h, dstp_h, hp1)

    b1r = jnp.broadcast_to(b1[None, :], (1, H))
    W2p = jnp.zeros((H, CP), jnp.float32).at[:, :C].set(W2)
    x1p, hp2 = pl.pallas_call(
        _mm2_body,
        grid=grid,
        in_specs=[_part_spec(H), _row_spec(H), _vec_spec(),
                  _full_spec(1, H), _full_spec(H, CP)],
        out_specs=[_row_spec(H), _row_spec(CP)],
        out_shape=[jax.ShapeDtypeStruct((NPAD, H), jnp.float32),
                   jax.ShapeDtypeStruct((NPAD, CP), jnp.float32)],
    )(s1, hp1, dinv, b1r, W2p)

    s2 = _scat_c(srcp_c, dstp_c, hp2)

    b2r = jnp.zeros((1, CP), jnp.float32).at[0, :C].set(b2)
    logits = pl.pallas_call(
        _fin_body,
        grid=grid,
        in_specs=[_part_spec(CP), _row_spec(CP), _vec_spec(), _full_spec(1, CP)],
        out_specs=_row_spec(CP),
        out_shape=jax.ShapeDtypeStruct((NPAD, CP), jnp.float32),
    )(s2, hp2, dinv, b2r)

    return (x1p[:N], logits[:N, :C])


# speedup vs baseline: 1.1376x; 1.1376x over previous
import functools

import jax
import jax.numpy as jnp
from jax.experimental import pallas as pl
from jax.experimental.pallas import tpu as pltpu
from jax.experimental.pallas import tpu_sc as plsc

N = 10000
E = 320000
D = 128
H = 128
C = 40
CP = 48

NC = 2
NS = 16
NW = NC * NS
NPAD = 10240
ROWS_W = NPAD // NW
ROWS_T = NPAD // NS
BLK = 512

EPW = 10240
EPAD = EPW * NW
PAD_ROW = N + 100

EBLK = 2000
NEB = E // EBLK

_SC_MESH = plsc.VectorSubcoreMesh(
    core_axis_name="c", subcore_axis_name="s", num_cores=NC, num_subcores=NS)


EPT = E // NW


@functools.partial(
    pl.kernel,
    out_type=jax.ShapeDtypeStruct((NC, NPAD), jnp.float32),
    mesh=_SC_MESH,
    scratch_types=[
        pltpu.VMEM((EPT,), jnp.int32),
        pltpu.VMEM((NPAD,), jnp.float32),
        pltpu.VMEM((NS, ROWS_T), jnp.float32),
        pltpu.VMEM_SHARED((NS, NPAD), jnp.float32),
    ],
    compiler_params=pltpu.CompilerParams(needs_layout_passes=False),
)
def _deg_kernel(dst_hbm, deg_hbm, dslice, hist, rbuf, stage):
    c = jax.lax.axis_index("c")
    s = jax.lax.axis_index("s")
    wid = s * NC + c

    zeros16 = jnp.zeros((16,), jnp.float32)

    def zbody(i, _):
        hist[pl.ds(i * 16, 16)] = zeros16
        return 0

    jax.lax.fori_loop(0, NPAD // 16, zbody, 0)

    off = pl.multiple_of(wid * EPT, 8)
    pltpu.sync_copy(dst_hbm.at[pl.ds(off, EPT)], dslice)

    ones16 = jnp.ones((16,), jnp.float32)

    def in_body(i, _):
        d16 = dslice[pl.ds(i * 16, 16)]
        plsc.addupdate_scatter(hist, [d16], ones16)
        return 0

    jax.lax.fori_loop(0, EPT // 16, in_body, 0)

    pltpu.sync_copy(hist, stage.at[s])
    plsc.subcore_barrier()

    row0 = s * ROWS_T
    pltpu.sync_copy(stage.at[:, pl.ds(row0, ROWS_T)], rbuf)

    def red_body(g, _):
        acc = rbuf[0, pl.ds(g * 16, 16)]
        for p in range(1, NS):
            acc = acc + rbuf[p, pl.ds(g * 16, 16)]
        hist[pl.ds(g * 16, 16)] = acc
        return 0

    jax.lax.fori_loop(0, ROWS_T // 16, red_body, 0)
    pltpu.sync_copy(hist.at[pl.ds(0, ROWS_T)],
                    deg_hbm.at[c, pl.ds(row0, ROWS_T)])


def _scat_body(srcp, dstp, hp, out_hbm,
               src0, src1, src2, src3, dst0, dst1, dst2, dst3,
               rows0, rows1, rows2, rows3, acc,
               gs0, gs1, gs2, gs3, ss0, ss1, ss2, ss3,
               is0, is1, is2, is3, *, F, CH, CNT0, CNT1):
    c = jax.lax.axis_index("c")
    s = jax.lax.axis_index("s")
    per_s = CNT0 + CNT1
    off = s * per_s + c * CNT0
    cnt = jnp.where(c == 0, CNT0, CNT1)

    srcv = [src0, src1, src2, src3]
    dstv = [dst0, dst1, dst2, dst3]
    rows = [rows0, rows1, rows2, rows3]
    gsem = [gs0, gs1, gs2, gs3]
    ssem = [ss0, ss1, ss2, ss3]
    isem = [is0, is1, is2, is3]

    def idx_start(j, t):
        pltpu.async_copy(srcp.at[off + j], srcv[t], isem[t])
        pltpu.async_copy(dstp.at[off + j], dstv[t], isem[t])

    def idx_wait(j, t):
        pltpu.make_async_copy(srcp.at[off + j], srcv[t], isem[t]).wait()
        pltpu.make_async_copy(dstp.at[off + j], dstv[t], isem[t]).wait()

    def gather_start(t):
        pltpu.async_copy(hp.at[srcv[t]], rows[t], gsem[t])

    def gather_wait(t):
        pltpu.make_async_copy(hp.at[srcv[t]], rows[t], gsem[t]).wait()

    def scat_start(t):
        pltpu.async_copy(rows[t], acc.at[dstv[t]], ssem[t], add=True)

    def scat_wait(t):
        pltpu.make_async_copy(rows[t], acc.at[dstv[t]], ssem[t]).wait()

    zeros16 = jnp.zeros((16,), jnp.float32)

    def zrow(i, _):
        def zcol(g, _):
            rows0[i, pl.ds(g * 16, 16)] = zeros16
            return 0
        jax.lax.fori_loop(0, F // 16, zcol, 0)
        return 0

    jax.lax.fori_loop(0, CH, zrow, 0)

    row0 = s * ROWS_T

    def zacc(b, _):
        pltpu.sync_copy(rows0, acc.at[pl.ds(row0 + b * CH, CH)])
        return 0

    jax.lax.fori_loop(0, ROWS_T // CH, zacc, 0)
    plsc.subcore_barrier()

    idx_start(0, 0)
    idx_start(1, 1)
    idx_wait(0, 0)
    gather_start(0)
    idx_wait(1, 1)
    gather_start(1)

    def body(k, _):
        for t in range(4):
            b = 4 * k + t
            u = (t + 2) % 4

            @pl.when(b < cnt)
            def _():
                gather_wait(t)
                scat_start(t)

            @pl.when((b >= 2) & (b < cnt + 2))
            def _():
                scat_wait(u)

            @pl.when(b + 2 < cnt)
            def _():
                idx_start(b + 2, u)
                idx_wait(b + 2, u)
                gather_start(u)

        return 0

    slots = max(CNT0, CNT1) + 2
    jax.lax.fori_loop(0, (slots + 3) // 4, body, 0)
    plsc.subcore_barrier()

    pltpu.sync_copy(acc.at[pl.ds(row0, ROWS_T)],
                    out_hbm.at[c, pl.ds(row0, ROWS_T)])


def _make_scat(F, CH, CNT0, CNT1):
    assert (CNT0 + CNT1) * NS * CH == EPAD
    return functools.partial(
        pl.kernel,
        out_type=jax.ShapeDtypeStruct((NC, NPAD, F), jnp.float32),
        mesh=_SC_MESH,
        scratch_types=(
            [pltpu.VMEM((CH,), jnp.int32) for _ in range(8)]
            + [pltpu.VMEM((CH, F), jnp.float32) for _ in range(4)]
            + [pltpu.VMEM_SHARED((NPAD, F), jnp.float32)]
            + [pltpu.SemaphoreType.DMA for _ in range(12)]
        ),
        compiler_params=pltpu.CompilerParams(use_tc_tiling_on_sc=False),
    )(functools.partial(_scat_body, F=F, CH=CH, CNT0=CNT0, CNT1=CNT1))


CH_H = 80
CH_C = 128
_scat_h = _make_scat(H, CH_H, 196, 60)
_scat_c = _make_scat(CP, CH_C, 96, 64)


def _mm1_body(x_ref, w1_ref, deg_ref, hp1_ref, dinv_ref):
    deg = deg_ref[0] + deg_ref[1] + 1.0
    dinv = jax.lax.rsqrt(deg)
    h = jnp.dot(x_ref[...], w1_ref[...], preferred_element_type=jnp.float32)
    hp1_ref[...] = h * dinv[:, None]
    dinv_ref[...] = dinv


def _mm2_body(s1_ref, hp1_ref, dinv_ref, b1_ref, w2_ref, x1_ref, hp2_ref):
    dinv = dinv_ref[...]
    x1 = (s1_ref[0] + s1_ref[1] + hp1_ref[...]) * dinv[:, None] + b1_ref[...]
    x1_ref[...] = x1
    x2 = jnp.maximum(x1, 0.0)
    h2 = jnp.dot(x2, w2_ref[...], preferred_element_type=jnp.float32)
    hp2_ref[...] = h2 * dinv[:, None]


def _fin_body(s2_ref, hp2_ref, dinv_ref, b2_ref, out_ref):
    dinv = dinv_ref[...]
    out_ref[...] = (s2_ref[0] + s2_ref[1] + hp2_ref[...]) * dinv[:, None] \
        + b2_ref[...]


def _row_spec(cols):
    return pl.BlockSpec((BLK, cols), lambda i: (i, 0))


def _part_spec(cols):
    return pl.BlockSpec((NC, BLK, cols), lambda i: (0, i, 0))


def _vec_spec():
    return pl.BlockSpec((BLK,), lambda i: (i,))


def _full_spec(r, co):
    return pl.BlockSpec((r, co), lambda i: (0, 0))


def kernel(last_e_emb, edge_index, W1, b1, W2, b2):
    src = edge_index[0]
    dst = edge_index[1]

    xp = jnp.zeros((NPAD, D), jnp.float32).at[:N].set(last_e_emb)

    pad = jnp.full((EPAD - E,), PAD_ROW, jnp.int32)
    srcf = jnp.concatenate([src, pad])
    dstf = jnp.concatenate([dst, pad])
    srcp_h = srcf.reshape(EPAD // CH_H, CH_H)
    dstp_h = dstf.reshape(EPAD // CH_H, CH_H)
    srcp_c = srcf.reshape(EPAD // CH_C, CH_C)
    dstp_c = dstf.reshape(EPAD // CH_C, CH_C)

    deg = _deg_kernel(dst)

    grid = (NPAD // BLK,)
    hp1, dinv = pl.pallas_call(
        _mm1_body,
        grid=grid,
        in_specs=[_row_spec(D), _full_spec(D, H),
                  pl.BlockSpec((NC, BLK), lambda i: (0, i))],
        out_specs=[_row_spec(H), _vec_spec()],
        out_shape=[jax.ShapeDtypeStruct((NPAD, H), jnp.float32),
                   jax.ShapeDtypeStruct((NPAD,), jnp.float32)],
    )(xp, W1, deg)

    s1 = _scat_h(srcp_h, dstp_h, hp1)

    b1r = jnp.broadcast_to(b1[None, :], (1, H))
    W2p = jnp.zeros((H, CP), jnp.float32).at[:, :C].set(W2)
    x1p, hp2 = pl.pallas_call(
        _mm2_body,
        grid=grid,
        in_specs=[_part_spec(H), _row_spec(H), _vec_spec(),
                  _full_spec(1, H), _full_spec(H, CP)],
        out_specs=[_row_spec(H), _row_spec(CP)],
        out_shape=[jax.ShapeDtypeStruct((NPAD, H), jnp.float32),
                   jax.ShapeDtypeStruct((NPAD, CP), jnp.float32)],
    )(s1, hp1, dinv, b1r, W2p)

    s2 = _scat_c(srcp_c, dstp_c, hp2)

    b2r = jnp.zeros((1, CP), jnp.float32).at[0, :C].set(b2)
    logits = pl.pallas_call(
        _fin_body,
        grid=grid,
        in_specs=[_part_spec(CP), _row_spec(CP), _vec_spec(), _full_spec(1, CP)],
        out_specs=_row_spec(CP),
        out_shape=jax.ShapeDtypeStruct((NPAD, CP), jnp.float32),
    )(s2, hp2, dinv, b2r)

    return (x1p[:N], logits[:N, :C])


# speedup vs baseline: 1.1635x; 1.0228x over previous
import functools

import jax
import jax.numpy as jnp
from jax.experimental import pallas as pl
from jax.experimental.pallas import tpu as pltpu
from jax.experimental.pallas import tpu_sc as plsc

N = 10000
E = 320000
D = 128
H = 128
C = 40
CP = 48

NC = 2
NS = 16
NW = NC * NS
NPAD = 10240
ROWS_W = NPAD // NW
ROWS_T = NPAD // NS
BLK = 512

EPW = 10240
EPAD = EPW * NW
PAD_ROW = N + 100

EBLK = 2000
NEB = E // EBLK

_SC_MESH = plsc.VectorSubcoreMesh(
    core_axis_name="c", subcore_axis_name="s", num_cores=NC, num_subcores=NS)


EPT = E // NW


@functools.partial(
    pl.kernel,
    out_type=jax.ShapeDtypeStruct((NC, NPAD), jnp.float32),
    mesh=_SC_MESH,
    scratch_types=[
        pltpu.VMEM((EPT,), jnp.int32),
        pltpu.VMEM((NPAD,), jnp.float32),
        pltpu.VMEM((NS, ROWS_T), jnp.float32),
        pltpu.VMEM_SHARED((NS, NPAD), jnp.float32),
    ],
    compiler_params=pltpu.CompilerParams(needs_layout_passes=False),
)
def _deg_kernel(dst_hbm, deg_hbm, dslice, hist, rbuf, stage):
    c = jax.lax.axis_index("c")
    s = jax.lax.axis_index("s")
    wid = s * NC + c

    zeros16 = jnp.zeros((16,), jnp.float32)

    def zbody(i, _):
        hist[pl.ds(i * 16, 16)] = zeros16
        return 0

    jax.lax.fori_loop(0, NPAD // 16, zbody, 0)

    off = pl.multiple_of(wid * EPT, 8)
    pltpu.sync_copy(dst_hbm.at[pl.ds(off, EPT)], dslice)

    ones16 = jnp.ones((16,), jnp.float32)

    def in_body(i, _):
        d16 = dslice[pl.ds(i * 16, 16)]
        plsc.addupdate_scatter(hist, [d16], ones16)
        return 0

    jax.lax.fori_loop(0, EPT // 16, in_body, 0)

    pltpu.sync_copy(hist, stage.at[s])
    plsc.subcore_barrier()

    row0 = s * ROWS_T
    pltpu.sync_copy(stage.at[:, pl.ds(row0, ROWS_T)], rbuf)

    def red_body(g, _):
        acc = rbuf[0, pl.ds(g * 16, 16)]
        for p in range(1, NS):
            acc = acc + rbuf[p, pl.ds(g * 16, 16)]
        hist[pl.ds(g * 16, 16)] = acc
        return 0

    jax.lax.fori_loop(0, ROWS_T // 16, red_body, 0)
    pltpu.sync_copy(hist.at[pl.ds(0, ROWS_T)],
                    deg_hbm.at[c, pl.ds(row0, ROWS_T)])


def _scat_body(srcp, dstp, hp, out_hbm,
               src0, src1, src2, src3, src4, src5, src6, src7,
               dst0, dst1, dst2, dst3, dst4, dst5, dst6, dst7,
               rows0, rows1, rows2, rows3, acc,
               gs0, gs1, gs2, gs3, ss0, ss1, ss2, ss3,
               is0, is1, is2, is3, is4, is5, is6, is7, *, F, CH, CNT0, CNT1):
    c = jax.lax.axis_index("c")
    s = jax.lax.axis_index("s")
    per_s = CNT0 + CNT1
    off = s * per_s + c * CNT0
    cnt = jnp.where(c == 0, CNT0, CNT1)

    srcv = [src0, src1, src2, src3, src4, src5, src6, src7]
    dstv = [dst0, dst1, dst2, dst3, dst4, dst5, dst6, dst7]
    rows = [rows0, rows1, rows2, rows3]
    gsem = [gs0, gs1, gs2, gs3]
    ssem = [ss0, ss1, ss2, ss3]
    isem = [is0, is1, is2, is3, is4, is5, is6, is7]

    def idx_start(j, q):
        pltpu.async_copy(srcp.at[off + j], srcv[q], isem[q])
        pltpu.async_copy(dstp.at[off + j], dstv[q], isem[q])

    def idx_wait(j, q):
        pltpu.make_async_copy(srcp.at[off + j], srcv[q], isem[q]).wait()
        pltpu.make_async_copy(dstp.at[off + j], dstv[q], isem[q]).wait()

    def gather_start(t, q):
        pltpu.async_copy(hp.at[srcv[q]], rows[t], gsem[t])

    def gather_wait(t, q):
        pltpu.make_async_copy(hp.at[srcv[q]], rows[t], gsem[t]).wait()

    def scat_start(t, q):
        pltpu.async_copy(rows[t], acc.at[dstv[q]], ssem[t], add=True)

    def scat_wait(t, q):
        pltpu.make_async_copy(rows[t], acc.at[dstv[q]], ssem[t]).wait()

    zeros16 = jnp.zeros((16,), jnp.float32)

    def zrow(i, _):
        def zcol(g, _):
            rows0[i, pl.ds(g * 16, 16)] = zeros16
            return 0
        jax.lax.fori_loop(0, F // 16, zcol, 0)
        return 0

    jax.lax.fori_loop(0, CH, zrow, 0)

    row0 = s * ROWS_T

    def zacc(b, _):
        pltpu.sync_copy(rows0, acc.at[pl.ds(row0 + b * CH, CH)])
        return 0

    jax.lax.fori_loop(0, ROWS_T // CH, zacc, 0)
    plsc.subcore_barrier()

    for q in range(4):
        idx_start(q, q)
    idx_wait(0, 0)
    gather_start(0, 0)
    idx_wait(1, 1)
    gather_start(1, 1)

    def body(k, _):
        for t8 in range(8):
            b = 8 * k + t8
            t = t8 % 4
            u = (t8 + 2) % 4
            q = t8
            qn = (t8 + 2) % 8
            qp = (t8 + 4) % 8

            @pl.when((b >= 1) & (b < cnt + 1))
            def _():
                scat_wait((t8 + 3) % 4, (t8 + 7) % 8)

            @pl.when(b < cnt)
            def _():
                gather_wait(t, q)
                scat_start(t, q)

            @pl.when(b + 2 < cnt)
            def _():
                idx_wait(b + 2, qn)
                gather_start(u, qn)

            @pl.when(b + 4 < cnt)
            def _():
                idx_start(b + 4, qp)

        return 0

    slots = max(CNT0, CNT1) + 2
    jax.lax.fori_loop(0, (slots + 7) // 8, body, 0)
    plsc.subcore_barrier()

    pltpu.sync_copy(acc.at[pl.ds(row0, ROWS_T)],
                    out_hbm.at[c, pl.ds(row0, ROWS_T)])


def _make_scat(F, CH, CNT0, CNT1):
    assert (CNT0 + CNT1) * NS * CH == EPAD
    return functools.partial(
        pl.kernel,
        out_type=jax.ShapeDtypeStruct((NC, NPAD, F), jnp.float32),
        mesh=_SC_MESH,
        scratch_types=(
            [pltpu.VMEM((CH,), jnp.int32) for _ in range(16)]
            + [pltpu.VMEM((CH, F), jnp.float32) for _ in range(4)]
            + [pltpu.VMEM_SHARED((NPAD, F), jnp.float32)]
            + [pltpu.SemaphoreType.DMA for _ in range(16)]
        ),
        compiler_params=pltpu.CompilerParams(use_tc_tiling_on_sc=False),
    )(functools.partial(_scat_body, F=F, CH=CH, CNT0=CNT0, CNT1=CNT1))


CH_H = 80
CH_C = 128
_scat_h = _make_scat(H, CH_H, 196, 60)
_scat_c = _make_scat(CP, CH_C, 96, 64)


def _mm1_body(x_ref, w1_ref, deg_ref, hp1_ref, dinv_ref):
    deg = deg_ref[0] + deg_ref[1] + 1.0
    dinv = jax.lax.rsqrt(deg)
    h = jnp.dot(x_ref[...], w1_ref[...], preferred_element_type=jnp.float32)
    hp1_ref[...] = h * dinv[:, None]
    dinv_ref[...] = dinv


def _mm2_body(s1_ref, hp1_ref, dinv_ref, b1_ref, w2_ref, x1_ref, hp2_ref):
    dinv = dinv_ref[...]
    x1 = (s1_ref[0] + s1_ref[1] + hp1_ref[...]) * dinv[:, None] + b1_ref[...]
    x1_ref[...] = x1
    x2 = jnp.maximum(x1, 0.0)
    h2 = jnp.dot(x2, w2_ref[...], preferred_element_type=jnp.float32)
    hp2_ref[...] = h2 * dinv[:, None]


def _fin_body(s2_ref, hp2_ref, dinv_ref, b2_ref, out_ref):
    dinv = dinv_ref[...]
    out_ref[...] = (s2_ref[0] + s2_ref[1] + hp2_ref[...]) * dinv[:, None] \
        + b2_ref[...]


def _row_spec(cols):
    return pl.BlockSpec((BLK, cols), lambda i: (i, 0))


def _part_spec(cols):
    return pl.BlockSpec((NC, BLK, cols), lambda i: (0, i, 0))


def _vec_spec():
    return pl.BlockSpec((BLK,), lambda i: (i,))


def _full_spec(r, co):
    return pl.BlockSpec((r, co), lambda i: (0, 0))


def kernel(last_e_emb, edge_index, W1, b1, W2, b2):
    src = edge_index[0]
    dst = edge_index[1]

    xp = jnp.zeros((NPAD, D), jnp.float32).at[:N].set(last_e_emb)

    pad = jnp.full((EPAD - E,), PAD_ROW, jnp.int32)
    srcf = jnp.concatenate([src, pad])
    dstf = jnp.concatenate([dst, pad])
    srcp_h = srcf.reshape(EPAD // CH_H, CH_H)
    dstp_h = dstf.reshape(EPAD // CH_H, CH_H)
    srcp_c = srcf.reshape(EPAD // CH_C, CH_C)
    dstp_c = dstf.reshape(EPAD // CH_C, CH_C)

    deg = _deg_kernel(dst)

    grid = (NPAD // BLK,)
    hp1, dinv = pl.pallas_call(
        _mm1_body,
        grid=grid,
        in_specs=[_row_spec(D), _full_spec(D, H),
                  pl.BlockSpec((NC, BLK), lambda i: (0, i))],
        out_specs=[_row_spec(H), _vec_spec()],
        out_shape=[jax.ShapeDtypeStruct((NPAD, H), jnp.float32),
                   jax.ShapeDtypeStruct((NPAD,), jnp.float32)],
    )(xp, W1, deg)

    s1 = _scat_h(srcp_h, dstp_h, hp1)

    b1r = jnp.broadcast_to(b1[None, :], (1, H))
    W2p = jnp.zeros((H, CP), jnp.float32).at[:, :C].set(W2)
    x1p, hp2 = pl.pallas_call(
        _mm2_body,
        grid=grid,
        in_specs=[_part_spec(H), _row_spec(H), _vec_spec(),
                  _full_spec(1, H), _full_spec(H, CP)],
        out_specs=[_row_spec(H), _row_spec(CP)],
        out_shape=[jax.ShapeDtypeStruct((NPAD, H), jnp.float32),
                   jax.ShapeDtypeStruct((NPAD, CP), jnp.float32)],
    )(s1, hp1, dinv, b1r, W2p)

    s2 = _scat_c(srcp_c, dstp_c, hp2)

    b2r = jnp.zeros((1, CP), jnp.float32).at[0, :C].set(b2)
    logits = pl.pallas_call(
        _fin_body,
        grid=grid,
        in_specs=[_part_spec(CP), _row_spec(CP), _vec_spec(), _full_spec(1, CP)],
        out_specs=_row_spec(CP),
        out_shape=jax.ShapeDtypeStruct((NPAD, CP), jnp.float32),
    )(s2, hp2, dinv, b2r)

    return (x1p[:N], logits[:N, :C])
